# Initial kernel scaffold; baseline (speedup 1.0000x reference)
import functools

import jax
import jax.numpy as jnp
from jax import lax
from jax.experimental import pallas as pl
from jax.experimental.pallas import tpu as pltpu
from jax.experimental.pallas import tpu_sc as plsc

N = 10000
D = 128
G = 64
E = 320000

NC = 2
NS = 16
NW = NC * NS
EPW = E // NW
CH = 125
NCHUNK = EPW // CH
ROWS_PER = N // NS
CW = 16

RB = 2000
NBLK = N // RB


def _sc_agg_body(with_counts, *refs):
    if with_counts:
        (h_hbm, src_hbm, dst_hbm, z_hbm, zc_hbm, one_hbm,
         agg_hbm, cnt_hbm, src_v, dst_v, rows_v, ones_v, acc_sh, cnt_sh,
         sem) = refs
    else:
        (h_hbm, src_hbm, dst_hbm, z_hbm,
         agg_hbm, src_v, dst_v, rows_v, acc_sh, sem) = refs
    c = lax.axis_index("c")
    s = lax.axis_index("s")
    wid = c * NS + s
    base = s * ROWS_PER

    pltpu.sync_copy(src_hbm.at[wid], src_v)
    pltpu.sync_copy(dst_hbm.at[wid], dst_v)
    pltpu.sync_copy(z_hbm, acc_sh.at[pl.ds(base, ROWS_PER)])
    if with_counts:
        pltpu.sync_copy(one_hbm, ones_v)
        pltpu.sync_copy(zc_hbm, cnt_sh.at[pl.ds(base, ROWS_PER)])
    plsc.subcore_barrier()

    def step(j, carry):
        pltpu.async_copy(h_hbm.at[src_v.at[j]], rows_v, sem).wait()
        pltpu.sync_copy(rows_v, acc_sh.at[dst_v.at[j]], add=True)
        if with_counts:
            pltpu.sync_copy(ones_v, cnt_sh.at[dst_v.at[j]], add=True)
        return carry

    lax.fori_loop(0, NCHUNK, step, 0)
    plsc.subcore_barrier()

    pltpu.sync_copy(acc_sh.at[pl.ds(base, ROWS_PER)],
                    agg_hbm.at[c, pl.ds(base, ROWS_PER)])
    if with_counts:
        pltpu.sync_copy(cnt_sh.at[pl.ds(base, ROWS_PER)],
                        cnt_hbm.at[c, pl.ds(base, ROWS_PER)])


def _make_sc_agg(with_counts):
    mesh = plsc.VectorSubcoreMesh(core_axis_name="c", subcore_axis_name="s")
    out_type = [jax.ShapeDtypeStruct((NC, N, D), jnp.float32)]
    scratch = [
        pltpu.VMEM((NCHUNK, CH), jnp.int32),
        pltpu.VMEM((NCHUNK, CH), jnp.int32),
        pltpu.VMEM((CH, D), jnp.float32),
    ]
    if with_counts:
        out_type.append(jax.ShapeDtypeStruct((NC, N, CW), jnp.float32))
        scratch.append(pltpu.VMEM((CH, CW), jnp.float32))
    scratch.append(pltpu.VMEM_SHARED((N, D), jnp.float32))
    if with_counts:
        scratch.append(pltpu.VMEM_SHARED((N, CW), jnp.float32))
    scratch.append(pltpu.SemaphoreType.DMA)
    return pl.kernel(
        functools.partial(_sc_agg_body, with_counts),
        out_type=tuple(out_type),
        mesh=mesh,
        scratch_types=tuple(scratch),
    )


def _combine_body(agg_ref, cnt_ref, h_ref, wl_ref, wr_ref, b_ref, o_ref):
    agg = agg_ref[0] + agg_ref[1]
    deg = cnt_ref[0, :, 0] + cnt_ref[1, :, 0]
    inv = 1.0 / jnp.maximum(deg, 1.0)
    o = (jnp.dot(agg * inv[:, None], wl_ref[...],
                 preferred_element_type=jnp.float32)
         + jnp.dot(h_ref[...], wr_ref[...],
                   preferred_element_type=jnp.float32)
         + b_ref[...])
    o_ref[...] = jnp.maximum(o, 0.0)


def _combine(agg, cnt, h, wl, wr, b):
    return pl.pallas_call(
        _combine_body,
        grid=(NBLK,),
        in_specs=[
            pl.BlockSpec((NC, RB, D), lambda i: (0, i, 0)),
            pl.BlockSpec((NC, RB, CW), lambda i: (0, i, 0)),
            pl.BlockSpec((RB, D), lambda i: (i, 0)),
            pl.BlockSpec((D, D), lambda i: (0, 0)),
            pl.BlockSpec((D, D), lambda i: (0, 0)),
            pl.BlockSpec((1, D), lambda i: (0, 0)),
        ],
        out_specs=pl.BlockSpec((RB, D), lambda i: (i, 0)),
        out_shape=jax.ShapeDtypeStruct((N, D), jnp.float32),
    )(agg, cnt, h, wl, wr, b)


def _final_body(agg_ref, cnt_ref, h_ref, wl_ref, wr_ref, b_ref, bat_ref,
                wp1_ref, bp1_ref, wp2_ref, bp2_ref, o_ref, pooled, gcnt):
    i = pl.program_id(0)

    @pl.when(i == 0)
    def _init():
        pooled[...] = jnp.zeros_like(pooled)
        gcnt[...] = jnp.zeros_like(gcnt)

    agg = agg_ref[0] + agg_ref[1]
    deg = cnt_ref[0, :, 0] + cnt_ref[1, :, 0]
    inv = 1.0 / jnp.maximum(deg, 1.0)
    h3 = jnp.maximum(
        jnp.dot(agg * inv[:, None], wl_ref[...],
                preferred_element_type=jnp.float32)
        + jnp.dot(h_ref[...], wr_ref[...], preferred_element_type=jnp.float32)
        + b_ref[...], 0.0)

    bvals = bat_ref[0, 0, :]
    gids = lax.broadcasted_iota(jnp.int32, (G, RB), 0)
    mask = jnp.where(bvals[None, :] == gids, 1.0, 0.0).astype(jnp.float32)
    pooled[...] += jnp.dot(mask, h3, preferred_element_type=jnp.float32)
    gcnt[...] += jnp.sum(mask, axis=1, keepdims=True)

    @pl.when(i == NBLK - 1)
    def _finish():
        pm = pooled[...] / jnp.maximum(gcnt[...], 1.0)
        z1 = jnp.maximum(
            jnp.dot(pm, wp1_ref[...], preferred_element_type=jnp.float32)
            + bp1_ref[...], 0.0)
        z2 = (jnp.dot(z1, wp2_ref[...], preferred_element_type=jnp.float32)
              + bp2_ref[...])
        nrm = jnp.sqrt(jnp.sum(z2 * z2, axis=1, keepdims=True))
        o_ref[...] = z2 / jnp.maximum(nrm, 1e-12)


def _final(agg, cnt, h, wl, wr, b, bat, wp1, bp1, wp2, bp2):
    return pl.pallas_call(
        _final_body,
        grid=(NBLK,),
        in_specs=[
            pl.BlockSpec((NC, RB, D), lambda i: (0, i, 0)),
            pl.BlockSpec((NC, RB, CW), lambda i: (0, i, 0)),
            pl.BlockSpec((RB, D), lambda i: (i, 0)),
            pl.BlockSpec((D, D), lambda i: (0, 0)),
            pl.BlockSpec((D, D), lambda i: (0, 0)),
            pl.BlockSpec((1, D), lambda i: (0, 0)),
            pl.BlockSpec((1, 1, RB), lambda i: (i, 0, 0)),
            pl.BlockSpec((D, D), lambda i: (0, 0)),
            pl.BlockSpec((1, D), lambda i: (0, 0)),
            pl.BlockSpec((D, D), lambda i: (0, 0)),
            pl.BlockSpec((1, D), lambda i: (0, 0)),
        ],
        out_specs=pl.BlockSpec((G, D), lambda i: (0, 0)),
        out_shape=jax.ShapeDtypeStruct((G, D), jnp.float32),
        scratch_shapes=[
            pltpu.VMEM((G, D), jnp.float32),
            pltpu.VMEM((G, 1), jnp.float32),
        ],
    )(agg, cnt, h, wl, wr, b, bat, wp1, bp1, wp2, bp2)


def kernel(x, edge_index, batch, Wl0, Wr0, b0, Wl1, Wr1, b1, Wl2, Wr2, b2,
           Wp1, bp1, Wp2, bp2):
    src = edge_index[0].reshape(NW, NCHUNK, CH)
    dst = edge_index[1].reshape(NW, NCHUNK, CH)
    zfeat = jnp.zeros((ROWS_PER, D), jnp.float32)
    zcnt = jnp.zeros((ROWS_PER, CW), jnp.float32)
    ones16 = jnp.ones((CH, CW), jnp.float32)
    bat3 = batch.reshape(NBLK, 1, RB)

    sc_agg0 = _make_sc_agg(True)
    sc_agg = _make_sc_agg(False)

    agg0, cnt = sc_agg0(x, src, dst, zfeat, zcnt, ones16)
    h1 = _combine(agg0, cnt, x, Wl0, Wr0, b0.reshape(1, D))
    agg1 = sc_agg(h1, src, dst, zfeat)
    h2 = _combine(agg1, cnt, h1, Wl1, Wr1, b1.reshape(1, D))
    agg2 = sc_agg(h2, src, dst, zfeat)
    return _final(agg2, cnt, h2, Wl2, Wr2, b2.reshape(1, D), bat3,
                  Wp1, bp1.reshape(1, D), Wp2, bp2.reshape(1, D))


# speedup vs baseline: 3.2788x; 3.2788x over previous
import functools

import jax
import jax.numpy as jnp
from jax import lax
from jax.experimental import pallas as pl
from jax.experimental.pallas import tpu as pltpu
from jax.experimental.pallas import tpu_sc as plsc

N = 10000
NP_ = 10240
D = 128
G = 64
E = 320000

NC = 2
NS = 16
NW = NC * NS
CH = 128
JB = 4
NBLOCK = 20
NCHUNK = JB * NBLOCK
EPAD = NW * NCHUNK * CH - E
ROWS_PER = NP_ // NS
HR = NP_ // D

RB = 2048
NBLK = NP_ // RB


def _sc_agg_body(h_hbm, src_hbm, dst_hbm, z_hbm,
                 agg_hbm, src_v, dst_v, rows_v, acc_sh, sem):
    c = lax.axis_index("c")
    s = lax.axis_index("s")
    wid = c * NS + s
    base = s * ROWS_PER

    pltpu.sync_copy(z_hbm, rows_v)
    for k in range(ROWS_PER // CH):
        pltpu.sync_copy(rows_v, acc_sh.at[pl.ds(base + k * CH, CH)])
    plsc.subcore_barrier()

    def block(jo, carry):
        pltpu.sync_copy(src_hbm.at[wid, pl.ds(jo * JB, JB)], src_v)
        pltpu.sync_copy(dst_hbm.at[wid, pl.ds(jo * JB, JB)], dst_v)
        for jj in range(JB):
            pltpu.async_copy(h_hbm.at[src_v.at[jj]], rows_v, sem).wait()
            pltpu.sync_copy(rows_v, acc_sh.at[dst_v.at[jj]], add=True)
        return carry

    lax.fori_loop(0, NBLOCK, block, 0)
    plsc.subcore_barrier()

    for k in range(ROWS_PER // CH):
        pltpu.sync_copy(acc_sh.at[pl.ds(base + k * CH, CH)], rows_v)
        pltpu.sync_copy(rows_v, agg_hbm.at[c, pl.ds(base + k * CH, CH)])


def _sc_deg_body(dst_hbm, one_hbm, z_hbm, cnt_hbm, dst_v, rows_v, acc_sh):
    c = lax.axis_index("c")
    s = lax.axis_index("s")
    wid = c * NS + s
    base = s * ROWS_PER

    pltpu.sync_copy(z_hbm, rows_v)
    for k in range(ROWS_PER // CH):
        pltpu.sync_copy(rows_v, acc_sh.at[pl.ds(base + k * CH, CH)])
    pltpu.sync_copy(one_hbm, rows_v)
    plsc.subcore_barrier()

    def block(jo, carry):
        pltpu.sync_copy(dst_hbm.at[wid, pl.ds(jo * JB, JB)], dst_v)
        for jj in range(JB):
            pltpu.sync_copy(rows_v, acc_sh.at[dst_v.at[jj]], add=True)
        return carry

    lax.fori_loop(0, NBLOCK, block, 0)
    plsc.subcore_barrier()

    for k in range(ROWS_PER // CH):
        pltpu.sync_copy(acc_sh.at[pl.ds(base + k * CH, CH)], rows_v)
        pltpu.sync_copy(rows_v, cnt_hbm.at[c, pl.ds(base + k * CH, CH)])


def _make_sc_agg():
    mesh = plsc.VectorSubcoreMesh(core_axis_name="c", subcore_axis_name="s")
    return pl.kernel(
        _sc_agg_body,
        out_type=(jax.ShapeDtypeStruct((NC, NP_, D), jnp.float32),),
        mesh=mesh,
        scratch_types=(
            pltpu.VMEM((JB, CH), jnp.int32),
            pltpu.VMEM((JB, CH), jnp.int32),
            pltpu.VMEM((CH, D), jnp.float32),
            pltpu.VMEM_SHARED((NP_, D), jnp.float32),
            pltpu.SemaphoreType.DMA,
        ),
    )


def _make_sc_deg():
    mesh = plsc.VectorSubcoreMesh(core_axis_name="c", subcore_axis_name="s")
    return pl.kernel(
        _sc_deg_body,
        out_type=(jax.ShapeDtypeStruct((NC, NP_, D), jnp.float32),),
        mesh=mesh,
        scratch_types=(
            pltpu.VMEM((JB, CH), jnp.int32),
            pltpu.VMEM((CH, D), jnp.float32),
            pltpu.VMEM_SHARED((NP_, D), jnp.float32),
        ),
    )


def _combine_body(agg_ref, cnt_ref, h_ref, wl_ref, wr_ref, b_ref, o_ref):
    agg = agg_ref[0] + agg_ref[1]
    inv = 1.0 / jnp.maximum(cnt_ref[0] + cnt_ref[1], 1.0)
    o = (jnp.dot(agg * inv, wl_ref[...],
                 preferred_element_type=jnp.float32)
         + jnp.dot(h_ref[...], wr_ref[...],
                   preferred_element_type=jnp.float32)
         + b_ref[...])
    o_ref[...] = jnp.maximum(o, 0.0)


def _combine(agg, cnt, h, wl, wr, b):
    return pl.pallas_call(
        _combine_body,
        grid=(NBLK,),
        in_specs=[
            pl.BlockSpec((NC, RB, D), lambda i: (0, i, 0)),
            pl.BlockSpec((NC, RB, D), lambda i: (0, i, 0)),
            pl.BlockSpec((RB, D), lambda i: (i, 0)),
            pl.BlockSpec((D, D), lambda i: (0, 0)),
            pl.BlockSpec((D, D), lambda i: (0, 0)),
            pl.BlockSpec((1, D), lambda i: (0, 0)),
        ],
        out_specs=pl.BlockSpec((RB, D), lambda i: (i, 0)),
        out_shape=jax.ShapeDtypeStruct((NP_, D), jnp.float32),
    )(agg, cnt, h, wl, wr, b)


def _final_body(agg_ref, cnt_ref, h_ref, wl_ref, wr_ref, b_ref, bat_ref,
                wp1_ref, bp1_ref, wp2_ref, bp2_ref, o_ref, pooled, gcnt):
    i = pl.program_id(0)

    @pl.when(i == 0)
    def _init():
        pooled[...] = jnp.zeros_like(pooled)
        gcnt[...] = jnp.zeros_like(gcnt)

    agg = agg_ref[0] + agg_ref[1]
    inv = 1.0 / jnp.maximum(cnt_ref[0] + cnt_ref[1], 1.0)
    h3 = jnp.maximum(
        jnp.dot(agg * inv, wl_ref[...],
                preferred_element_type=jnp.float32)
        + jnp.dot(h_ref[...], wr_ref[...], preferred_element_type=jnp.float32)
        + b_ref[...], 0.0)

    bvals = bat_ref[0, 0, :]
    gids = lax.broadcasted_iota(jnp.int32, (G, RB), 0)
    mask = jnp.where(bvals[None, :] == gids, 1.0, 0.0).astype(jnp.float32)
    pooled[...] += jnp.dot(mask, h3, preferred_element_type=jnp.float32)
    gcnt[...] += jnp.sum(mask, axis=1, keepdims=True)

    @pl.when(i == NBLK - 1)
    def _finish():
        pm = pooled[...] / jnp.maximum(gcnt[...], 1.0)
        z1 = jnp.maximum(
            jnp.dot(pm, wp1_ref[...], preferred_element_type=jnp.float32)
            + bp1_ref[...], 0.0)
        z2 = (jnp.dot(z1, wp2_ref[...], preferred_element_type=jnp.float32)
              + bp2_ref[...])
        nrm = jnp.sqrt(jnp.sum(z2 * z2, axis=1, keepdims=True))
        o_ref[...] = z2 / jnp.maximum(nrm, 1e-12)


def _final(agg, cnt, h, wl, wr, b, bat, wp1, bp1, wp2, bp2):
    return pl.pallas_call(
        _final_body,
        grid=(NBLK,),
        in_specs=[
            pl.BlockSpec((NC, RB, D), lambda i: (0, i, 0)),
            pl.BlockSpec((NC, RB, D), lambda i: (0, i, 0)),
            pl.BlockSpec((RB, D), lambda i: (i, 0)),
            pl.BlockSpec((D, D), lambda i: (0, 0)),
            pl.BlockSpec((D, D), lambda i: (0, 0)),
            pl.BlockSpec((1, D), lambda i: (0, 0)),
            pl.BlockSpec((1, 1, RB), lambda i: (i, 0, 0)),
            pl.BlockSpec((D, D), lambda i: (0, 0)),
            pl.BlockSpec((1, D), lambda i: (0, 0)),
            pl.BlockSpec((D, D), lambda i: (0, 0)),
            pl.BlockSpec((1, D), lambda i: (0, 0)),
        ],
        out_specs=pl.BlockSpec((G, D), lambda i: (0, 0)),
        out_shape=jax.ShapeDtypeStruct((G, D), jnp.float32),
        scratch_shapes=[
            pltpu.VMEM((G, D), jnp.float32),
            pltpu.VMEM((G, 1), jnp.float32),
        ],
    )(agg, cnt, h, wl, wr, b, bat, wp1, bp1, wp2, bp2)


def kernel(x, edge_index, batch, Wl0, Wr0, b0, Wl1, Wr1, b1, Wl2, Wr2, b2,
           Wp1, bp1, Wp2, bp2):
    src = jnp.concatenate(
        [edge_index[0], jnp.zeros((EPAD,), jnp.int32)]).reshape(NW, NCHUNK, CH)
    dst = jnp.concatenate(
        [edge_index[1], jnp.full((EPAD,), N, jnp.int32)]).reshape(NW, NCHUNK, CH)
    xp = jnp.pad(x, ((0, NP_ - N), (0, 0)))
    zfeat = jnp.zeros((CH, D), jnp.float32)
    ones128 = jnp.ones((CH, D), jnp.float32)
    bat3 = jnp.concatenate(
        [batch, jnp.full((NP_ - N,), -1, jnp.int32)]).reshape(NBLK, 1, RB)

    sc_agg = _make_sc_agg()

    (cnt,) = _make_sc_deg()(dst, ones128, zfeat)
    (agg0,) = sc_agg(xp, src, dst, zfeat)
    h1 = _combine(agg0, cnt, xp, Wl0, Wr0, b0.reshape(1, D))
    (agg1,) = sc_agg(h1, src, dst, zfeat)
    h2 = _combine(agg1, cnt, h1, Wl1, Wr1, b1.reshape(1, D))
    (agg2,) = sc_agg(h2, src, dst, zfeat)
    return _final(agg2, cnt, h2, Wl2, Wr2, b2.reshape(1, D), bat3,
                  Wp1, bp1.reshape(1, D), Wp2, bp2.reshape(1, D))


# speedup vs baseline: 3.4411x; 1.0495x over previous
import functools

import jax
import jax.numpy as jnp
from jax import lax
from jax.experimental import pallas as pl
from jax.experimental.pallas import tpu as pltpu
from jax.experimental.pallas import tpu_sc as plsc

N = 10000
NP_ = 10240
D = 128
G = 64
E = 320000

NC = 2
NS = 16
NW = NC * NS
CH = 128
JB = 8
NBLOCK = 10
NCHUNK = JB * NBLOCK
EPAD = NW * NCHUNK * CH - E
KTOT = 160
KC0 = 112
APAD = NS * KTOT * CH - E
ROWS_PER = NP_ // NS
HR = NP_ // D

RB = 2048
NBLK = NP_ // RB


def _sc_agg_body(h_hbm, src_hbm, dst_hbm, z_hbm,
                 agg_hbm, src_v, dst_v, rows_v, acc_sh, sem, ssem):
    c = lax.axis_index("c")
    s = lax.axis_index("s")
    koff = c * (KTOT - KC0)
    nblocks = ((KTOT - KC0) // JB) + ((KC0 // JB) - ((KTOT - KC0) // JB)) * c
    base = s * ROWS_PER

    pltpu.sync_copy(z_hbm, rows_v.at[0])
    for k in range(ROWS_PER // CH):
        pltpu.sync_copy(rows_v.at[0], acc_sh.at[pl.ds(base + k * CH, CH)])
    plsc.subcore_barrier()

    def _wait_scatter(p):
        pltpu.make_async_copy(
            rows_v.at[p], acc_sh.at[dst_v.at[0]], ssem[p]).wait()

    def block(jo, carry):
        pltpu.sync_copy(src_hbm.at[s, pl.ds(koff + jo * JB, JB)], src_v)
        pltpu.sync_copy(dst_hbm.at[s, pl.ds(koff + jo * JB, JB)], dst_v)
        @pl.when(jo > 0)
        def _drain0():
            _wait_scatter(0)

        pltpu.async_copy(h_hbm.at[src_v.at[0]], rows_v.at[0], sem[0])
        for jj in range(JB):
            p = jj % 2
            q = 1 - p
            pltpu.make_async_copy(
                h_hbm.at[src_v.at[jj]], rows_v.at[p], sem[p]).wait()
            if jj + 1 < JB:
                if jj == 0:
                    @pl.when(jo > 0)
                    def _drain1():
                        _wait_scatter(1)
                else:
                    _wait_scatter(q)
                pltpu.async_copy(h_hbm.at[src_v.at[jj + 1]],
                                 rows_v.at[q], sem[q])
            pltpu.async_copy(rows_v.at[p], acc_sh.at[dst_v.at[jj]], ssem[p],
                             add=True)
        return carry

    lax.fori_loop(0, nblocks, block, 0)
    _wait_scatter(0)
    _wait_scatter(1)
    plsc.subcore_barrier()

    for k in range(ROWS_PER // CH):
        p = k % 2
        if k >= 2:
            pltpu.make_async_copy(
                rows_v.at[p], agg_hbm.at[c, pl.ds(base, CH)], sem[p]).wait()
        pltpu.sync_copy(acc_sh.at[pl.ds(base + k * CH, CH)], rows_v.at[p])
        pltpu.async_copy(rows_v.at[p],
                         agg_hbm.at[c, pl.ds(base + k * CH, CH)], sem[p])
    for p in range(2):
        pltpu.make_async_copy(
            rows_v.at[p], agg_hbm.at[c, pl.ds(base, CH)], sem[p]).wait()


def _sc_deg_body(dst_hbm, one_hbm, z_hbm, cnt_hbm, dst_v, rows_v, acc_sh,
                 dsem):
    c = lax.axis_index("c")
    s = lax.axis_index("s")
    wid = c * NS + s
    base = s * ROWS_PER

    pltpu.sync_copy(z_hbm, rows_v)
    for k in range(ROWS_PER // CH):
        pltpu.sync_copy(rows_v, acc_sh.at[pl.ds(base + k * CH, CH)])
    pltpu.sync_copy(one_hbm, rows_v)
    plsc.subcore_barrier()

    def block(jo, carry):
        pltpu.sync_copy(dst_hbm.at[wid, pl.ds(jo * JB, JB)], dst_v)
        for jj in range(JB):
            pltpu.async_copy(rows_v, acc_sh.at[dst_v.at[jj]], dsem, add=True)
        for jj in range(JB):
            pltpu.make_async_copy(rows_v, acc_sh.at[dst_v.at[0]], dsem).wait()
        return carry

    lax.fori_loop(0, NBLOCK, block, 0)
    plsc.subcore_barrier()

    for k in range(ROWS_PER // CH):
        pltpu.sync_copy(acc_sh.at[pl.ds(base + k * CH, CH)], rows_v)
        pltpu.sync_copy(rows_v, cnt_hbm.at[c, pl.ds(base + k * CH, CH)])


def _make_sc_agg():
    mesh = plsc.VectorSubcoreMesh(core_axis_name="c", subcore_axis_name="s")
    return pl.kernel(
        _sc_agg_body,
        out_type=(jax.ShapeDtypeStruct((NC, NP_, D), jnp.float32),),
        mesh=mesh,
        scratch_types=(
            pltpu.VMEM((JB, CH), jnp.int32),
            pltpu.VMEM((JB, CH), jnp.int32),
            pltpu.VMEM((2, CH, D), jnp.float32),
            pltpu.VMEM_SHARED((NP_, D), jnp.float32),
            (pltpu.SemaphoreType.DMA, pltpu.SemaphoreType.DMA),
            (pltpu.SemaphoreType.DMA, pltpu.SemaphoreType.DMA),
        ),
    )


def _make_sc_deg():
    mesh = plsc.VectorSubcoreMesh(core_axis_name="c", subcore_axis_name="s")
    return pl.kernel(
        _sc_deg_body,
        out_type=(jax.ShapeDtypeStruct((NC, NP_, D), jnp.float32),),
        mesh=mesh,
        scratch_types=(
            pltpu.VMEM((JB, CH), jnp.int32),
            pltpu.VMEM((CH, D), jnp.float32),
            pltpu.VMEM_SHARED((NP_, D), jnp.float32),
            pltpu.SemaphoreType.DMA,
        ),
    )


def _combine_body(agg_ref, cnt_ref, h_ref, wl_ref, wr_ref, b_ref, o_ref):
    agg = agg_ref[0] + agg_ref[1]
    inv = 1.0 / jnp.maximum(cnt_ref[0] + cnt_ref[1], 1.0)
    o = (jnp.dot(agg * inv, wl_ref[...],
                 preferred_element_type=jnp.float32)
         + jnp.dot(h_ref[...], wr_ref[...],
                   preferred_element_type=jnp.float32)
         + b_ref[...])
    o_ref[...] = jnp.maximum(o, 0.0)


def _combine(agg, cnt, h, wl, wr, b):
    return pl.pallas_call(
        _combine_body,
        grid=(NBLK,),
        in_specs=[
            pl.BlockSpec((NC, RB, D), lambda i: (0, i, 0)),
            pl.BlockSpec((NC, RB, D), lambda i: (0, i, 0)),
            pl.BlockSpec((RB, D), lambda i: (i, 0)),
            pl.BlockSpec((D, D), lambda i: (0, 0)),
            pl.BlockSpec((D, D), lambda i: (0, 0)),
            pl.BlockSpec((1, D), lambda i: (0, 0)),
        ],
        out_specs=pl.BlockSpec((RB, D), lambda i: (i, 0)),
        out_shape=jax.ShapeDtypeStruct((NP_, D), jnp.float32),
    )(agg, cnt, h, wl, wr, b)


def _final_body(agg_ref, cnt_ref, h_ref, wl_ref, wr_ref, b_ref, bat_ref,
                wp1_ref, bp1_ref, wp2_ref, bp2_ref, o_ref, pooled, gcnt):
    i = pl.program_id(0)

    @pl.when(i == 0)
    def _init():
        pooled[...] = jnp.zeros_like(pooled)
        gcnt[...] = jnp.zeros_like(gcnt)

    agg = agg_ref[0] + agg_ref[1]
    inv = 1.0 / jnp.maximum(cnt_ref[0] + cnt_ref[1], 1.0)
    h3 = jnp.maximum(
        jnp.dot(agg * inv, wl_ref[...],
                preferred_element_type=jnp.float32)
        + jnp.dot(h_ref[...], wr_ref[...], preferred_element_type=jnp.float32)
        + b_ref[...], 0.0)

    bvals = bat_ref[0, 0, :]
    gids = lax.broadcasted_iota(jnp.int32, (G, RB), 0)
    mask = jnp.where(bvals[None, :] == gids, 1.0, 0.0).astype(jnp.float32)
    pooled[...] += jnp.dot(mask, h3, preferred_element_type=jnp.float32)
    gcnt[...] += jnp.sum(mask, axis=1, keepdims=True)

    @pl.when(i == NBLK - 1)
    def _finish():
        pm = pooled[...] / jnp.maximum(gcnt[...], 1.0)
        z1 = jnp.maximum(
            jnp.dot(pm, wp1_ref[...], preferred_element_type=jnp.float32)
            + bp1_ref[...], 0.0)
        z2 = (jnp.dot(z1, wp2_ref[...], preferred_element_type=jnp.float32)
              + bp2_ref[...])
        nrm = jnp.sqrt(jnp.sum(z2 * z2, axis=1, keepdims=True))
        o_ref[...] = z2 / jnp.maximum(nrm, 1e-12)


def _final(agg, cnt, h, wl, wr, b, bat, wp1, bp1, wp2, bp2):
    return pl.pallas_call(
        _final_body,
        grid=(NBLK,),
        in_specs=[
            pl.BlockSpec((NC, RB, D), lambda i: (0, i, 0)),
            pl.BlockSpec((NC, RB, D), lambda i: (0, i, 0)),
            pl.BlockSpec((RB, D), lambda i: (i, 0)),
            pl.BlockSpec((D, D), lambda i: (0, 0)),
            pl.BlockSpec((D, D), lambda i: (0, 0)),
            pl.BlockSpec((1, D), lambda i: (0, 0)),
            pl.BlockSpec((1, 1, RB), lambda i: (i, 0, 0)),
            pl.BlockSpec((D, D), lambda i: (0, 0)),
            pl.BlockSpec((1, D), lambda i: (0, 0)),
            pl.BlockSpec((D, D), lambda i: (0, 0)),
            pl.BlockSpec((1, D), lambda i: (0, 0)),
        ],
        out_specs=pl.BlockSpec((G, D), lambda i: (0, 0)),
        out_shape=jax.ShapeDtypeStruct((G, D), jnp.float32),
        scratch_shapes=[
            pltpu.VMEM((G, D), jnp.float32),
            pltpu.VMEM((G, 1), jnp.float32),
        ],
    )(agg, cnt, h, wl, wr, b, bat, wp1, bp1, wp2, bp2)


def kernel(x, edge_index, batch, Wl0, Wr0, b0, Wl1, Wr1, b1, Wl2, Wr2, b2,
           Wp1, bp1, Wp2, bp2):
    src = jnp.concatenate(
        [edge_index[0], jnp.zeros((APAD,), jnp.int32)]).reshape(NS, KTOT, CH)
    dst = jnp.concatenate(
        [edge_index[1], jnp.full((APAD,), N, jnp.int32)]).reshape(NS, KTOT, CH)
    dste = jnp.concatenate(
        [edge_index[1], jnp.full((EPAD,), N, jnp.int32)]).reshape(NW, NCHUNK, CH)
    xp = jnp.pad(x, ((0, NP_ - N), (0, 0)))
    zfeat = jnp.zeros((CH, D), jnp.float32)
    ones128 = jnp.ones((CH, D), jnp.float32)
    bat3 = jnp.concatenate(
        [batch, jnp.full((NP_ - N,), -1, jnp.int32)]).reshape(NBLK, 1, RB)

    sc_agg = _make_sc_agg()

    (cnt,) = _make_sc_deg()(dste, ones128, zfeat)
    (agg0,) = sc_agg(xp, src, dst, zfeat)
    h1 = _combine(agg0, cnt, xp, Wl0, Wr0, b0.reshape(1, D))
    (agg1,) = sc_agg(h1, src, dst, zfeat)
    h2 = _combine(agg1, cnt, h1, Wl1, Wr1, b1.reshape(1, D))
    (agg2,) = sc_agg(h2, src, dst, zfeat)
    return _final(agg2, cnt, h2, Wl2, Wr2, b2.reshape(1, D), bat3,
                  Wp1, bp1.reshape(1, D), Wp2, bp2.reshape(1, D))


# speedup vs baseline: 4.2219x; 1.2269x over previous
import functools

import jax
import jax.numpy as jnp
from jax import lax
from jax.experimental import pallas as pl
from jax.experimental.pallas import tpu as pltpu
from jax.experimental.pallas import tpu_sc as plsc

N = 10000
NP_ = 10240
D = 128
G = 64
E = 320000

NC = 2
NS = 16
NW = NC * NS
CH = 128
JB = 8
NBLOCK = 10
NCHUNK = JB * NBLOCK
EPAD = NW * NCHUNK * CH - E
KTOT = 160
KC0 = 120
APAD = NS * KTOT * CH - E
ROWS_PER = NP_ // NS
HR = NP_ // D

RB = 2048
NBLK = NP_ // RB


def _sc_agg_body(h_hbm, src_hbm, dst_hbm, z_hbm,
                 agg_hbm, src_v, dst_v, rows_v, acc_sh, sem, ssem):
    c = lax.axis_index("c")
    s = lax.axis_index("s")
    koff = c * KC0
    nblocks = (KC0 // JB) - ((KC0 // JB) - ((KTOT - KC0) // JB)) * c
    base = s * ROWS_PER

    pltpu.sync_copy(z_hbm, rows_v.at[0])
    for k in range(ROWS_PER // CH):
        pltpu.sync_copy(rows_v.at[0], acc_sh.at[pl.ds(base + k * CH, CH)])
    plsc.subcore_barrier()

    def _wait_scatter(p):
        pltpu.make_async_copy(
            rows_v.at[p], acc_sh.at[dst_v.at[0]], ssem[p]).wait()

    def block(jo, carry):
        pltpu.sync_copy(src_hbm.at[s, pl.ds(koff + jo * JB, JB)], src_v)
        pltpu.sync_copy(dst_hbm.at[s, pl.ds(koff + jo * JB, JB)], dst_v)
        @pl.when(jo > 0)
        def _drain0():
            _wait_scatter(0)

        pltpu.async_copy(h_hbm.at[src_v.at[0]], rows_v.at[0], sem[0])
        for jj in range(JB):
            p = jj % 2
            q = 1 - p
            pltpu.make_async_copy(
                h_hbm.at[src_v.at[jj]], rows_v.at[p], sem[p]).wait()
            if jj + 1 < JB:
                if jj == 0:
                    @pl.when(jo > 0)
                    def _drain1():
                        _wait_scatter(1)
                else:
                    _wait_scatter(q)
                pltpu.async_copy(h_hbm.at[src_v.at[jj + 1]],
                                 rows_v.at[q], sem[q])
            pltpu.async_copy(rows_v.at[p], acc_sh.at[dst_v.at[jj]], ssem[p],
                             add=True)
        return carry

    lax.fori_loop(0, nblocks, block, 0)
    _wait_scatter(0)
    _wait_scatter(1)
    plsc.subcore_barrier()

    for k in range(ROWS_PER // CH):
        p = k % 2
        if k >= 2:
            pltpu.make_async_copy(
                rows_v.at[p], agg_hbm.at[c, pl.ds(base, CH)], sem[p]).wait()
        pltpu.sync_copy(acc_sh.at[pl.ds(base + k * CH, CH)], rows_v.at[p])
        pltpu.async_copy(rows_v.at[p],
                         agg_hbm.at[c, pl.ds(base + k * CH, CH)], sem[p])
    for p in range(2):
        pltpu.make_async_copy(
            rows_v.at[p], agg_hbm.at[c, pl.ds(base, CH)], sem[p]).wait()


def _sc_deg_body(dst_hbm, one_hbm, z_hbm, cnt_hbm, dst_v, rows_v, acc_sh,
                 dsem):
    c = lax.axis_index("c")
    s = lax.axis_index("s")
    wid = c * NS + s
    base = s * ROWS_PER

    pltpu.sync_copy(z_hbm, rows_v)
    for k in range(ROWS_PER // CH):
        pltpu.sync_copy(rows_v, acc_sh.at[pl.ds(base + k * CH, CH)])
    pltpu.sync_copy(one_hbm, rows_v)
    plsc.subcore_barrier()

    def block(jo, carry):
        pltpu.sync_copy(dst_hbm.at[wid, pl.ds(jo * JB, JB)], dst_v)
        for jj in range(JB):
            pltpu.async_copy(rows_v, acc_sh.at[dst_v.at[jj]], dsem, add=True)
        for jj in range(JB):
            pltpu.make_async_copy(rows_v, acc_sh.at[dst_v.at[0]], dsem).wait()
        return carry

    lax.fori_loop(0, NBLOCK, block, 0)
    plsc.subcore_barrier()

    for k in range(ROWS_PER // CH):
        pltpu.sync_copy(acc_sh.at[pl.ds(base + k * CH, CH)], rows_v)
        pltpu.sync_copy(rows_v, cnt_hbm.at[c, pl.ds(base + k * CH, CH)])


def _make_sc_agg():
    mesh = plsc.VectorSubcoreMesh(core_axis_name="c", subcore_axis_name="s")
    return pl.kernel(
        _sc_agg_body,
        out_type=(jax.ShapeDtypeStruct((NC, NP_, D), jnp.float32),),
        mesh=mesh,
        scratch_types=(
            pltpu.VMEM((JB, CH), jnp.int32),
            pltpu.VMEM((JB, CH), jnp.int32),
            pltpu.VMEM((2, CH, D), jnp.float32),
            pltpu.VMEM_SHARED((NP_, D), jnp.float32),
            (pltpu.SemaphoreType.DMA, pltpu.SemaphoreType.DMA),
            (pltpu.SemaphoreType.DMA, pltpu.SemaphoreType.DMA),
        ),
    )


def _make_sc_deg():
    mesh = plsc.VectorSubcoreMesh(core_axis_name="c", subcore_axis_name="s")
    return pl.kernel(
        _sc_deg_body,
        out_type=(jax.ShapeDtypeStruct((NC, NP_, D), jnp.float32),),
        mesh=mesh,
        scratch_types=(
            pltpu.VMEM((JB, CH), jnp.int32),
            pltpu.VMEM((CH, D), jnp.float32),
            pltpu.VMEM_SHARED((NP_, D), jnp.float32),
            pltpu.SemaphoreType.DMA,
        ),
    )


def _combine_body(agg_ref, cnt_ref, h_ref, wl_ref, wr_ref, b_ref, o_ref):
    agg = agg_ref[0] + agg_ref[1]
    inv = 1.0 / jnp.maximum(cnt_ref[0] + cnt_ref[1], 1.0)
    o = (jnp.dot(agg * inv, wl_ref[...],
                 preferred_element_type=jnp.float32)
         + jnp.dot(h_ref[...], wr_ref[...],
                   preferred_element_type=jnp.float32)
         + b_ref[...])
    o_ref[...] = jnp.maximum(o, 0.0)


def _combine(agg, cnt, h, wl, wr, b):
    return pl.pallas_call(
        _combine_body,
        grid=(NBLK,),
        in_specs=[
            pl.BlockSpec((NC, RB, D), lambda i: (0, i, 0)),
            pl.BlockSpec((NC, RB, D), lambda i: (0, i, 0)),
            pl.BlockSpec((RB, D), lambda i: (i, 0)),
            pl.BlockSpec((D, D), lambda i: (0, 0)),
            pl.BlockSpec((D, D), lambda i: (0, 0)),
            pl.BlockSpec((1, D), lambda i: (0, 0)),
        ],
        out_specs=pl.BlockSpec((RB, D), lambda i: (i, 0)),
        out_shape=jax.ShapeDtypeStruct((NP_, D), jnp.float32),
    )(agg, cnt, h, wl, wr, b)


def _final_body(agg_ref, cnt_ref, h_ref, wl_ref, wr_ref, b_ref, bat_ref,
                wp1_ref, bp1_ref, wp2_ref, bp2_ref, o_ref, pooled, gcnt):
    i = pl.program_id(0)

    @pl.when(i == 0)
    def _init():
        pooled[...] = jnp.zeros_like(pooled)
        gcnt[...] = jnp.zeros_like(gcnt)

    agg = agg_ref[0] + agg_ref[1]
    inv = 1.0 / jnp.maximum(cnt_ref[0] + cnt_ref[1], 1.0)
    h3 = jnp.maximum(
        jnp.dot(agg * inv, wl_ref[...],
                preferred_element_type=jnp.float32)
        + jnp.dot(h_ref[...], wr_ref[...], preferred_element_type=jnp.float32)
        + b_ref[...], 0.0)

    bvals = bat_ref[0, 0, :]
    gids = lax.broadcasted_iota(jnp.int32, (G, RB), 0)
    mask = jnp.where(bvals[None, :] == gids, 1.0, 0.0).astype(jnp.float32)
    pooled[...] += jnp.dot(mask, h3, preferred_element_type=jnp.float32)
    gcnt[...] += jnp.sum(mask, axis=1, keepdims=True)

    @pl.when(i == NBLK - 1)
    def _finish():
        pm = pooled[...] / jnp.maximum(gcnt[...], 1.0)
        z1 = jnp.maximum(
            jnp.dot(pm, wp1_ref[...], preferred_element_type=jnp.float32)
            + bp1_ref[...], 0.0)
        z2 = (jnp.dot(z1, wp2_ref[...], preferred_element_type=jnp.float32)
              + bp2_ref[...])
        nrm = jnp.sqrt(jnp.sum(z2 * z2, axis=1, keepdims=True))
        o_ref[...] = z2 / jnp.maximum(nrm, 1e-12)


def _final(agg, cnt, h, wl, wr, b, bat, wp1, bp1, wp2, bp2):
    return pl.pallas_call(
        _final_body,
        grid=(NBLK,),
        in_specs=[
            pl.BlockSpec((NC, RB, D), lambda i: (0, i, 0)),
            pl.BlockSpec((NC, RB, D), lambda i: (0, i, 0)),
            pl.BlockSpec((RB, D), lambda i: (i, 0)),
            pl.BlockSpec((D, D), lambda i: (0, 0)),
            pl.BlockSpec((D, D), lambda i: (0, 0)),
            pl.BlockSpec((1, D), lambda i: (0, 0)),
            pl.BlockSpec((1, 1, RB), lambda i: (i, 0, 0)),
            pl.BlockSpec((D, D), lambda i: (0, 0)),
            pl.BlockSpec((1, D), lambda i: (0, 0)),
            pl.BlockSpec((D, D), lambda i: (0, 0)),
            pl.BlockSpec((1, D), lambda i: (0, 0)),
        ],
        out_specs=pl.BlockSpec((G, D), lambda i: (0, 0)),
        out_shape=jax.ShapeDtypeStruct((G, D), jnp.float32),
        scratch_shapes=[
            pltpu.VMEM((G, D), jnp.float32),
            pltpu.VMEM((G, 1), jnp.float32),
        ],
    )(agg, cnt, h, wl, wr, b, bat, wp1, bp1, wp2, bp2)


def kernel(x, edge_index, batch, Wl0, Wr0, b0, Wl1, Wr1, b1, Wl2, Wr2, b2,
           Wp1, bp1, Wp2, bp2):
    src = jnp.concatenate(
        [edge_index[0], jnp.zeros((APAD,), jnp.int32)]).reshape(NS, KTOT, CH)
    dst = jnp.concatenate(
        [edge_index[1], jnp.full((APAD,), N, jnp.int32)]).reshape(NS, KTOT, CH)
    dste = jnp.concatenate(
        [edge_index[1], jnp.full((EPAD,), N, jnp.int32)]).reshape(NW, NCHUNK, CH)
    xp = jnp.pad(x, ((0, NP_ - N), (0, 0)))
    zfeat = jnp.zeros((CH, D), jnp.float32)
    ones128 = jnp.ones((CH, D), jnp.float32)
    bat3 = jnp.concatenate(
        [batch, jnp.full((NP_ - N,), -1, jnp.int32)]).reshape(NBLK, 1, RB)

    sc_agg = _make_sc_agg()

    (cnt,) = _make_sc_deg()(dste, ones128, zfeat)
    (agg0,) = sc_agg(xp, src, dst, zfeat)
    h1 = _combine(agg0, cnt, xp, Wl0, Wr0, b0.reshape(1, D))
    (agg1,) = sc_agg(h1, src, dst, zfeat)
    h2 = _combine(agg1, cnt, h1, Wl1, Wr1, b1.reshape(1, D))
    (agg2,) = sc_agg(h2, src, dst, zfeat)
    return _final(agg2, cnt, h2, Wl2, Wr2, b2.reshape(1, D), bat3,
                  Wp1, bp1.reshape(1, D), Wp2, bp2.reshape(1, D))
